# Initial kernel scaffold; baseline (speedup 1.0000x reference)
import jax
import jax.numpy as jnp
from jax.experimental import pallas as pl


def kernel(x, edge_index, W, a_src, a_dst):
    raise NotImplementedError("write your pallas kernel here")


# speedup vs baseline: 14.2226x; 14.2226x over previous
import dataclasses
import functools

import jax
import jax.numpy as jnp
from jax import lax
from jax.experimental import pallas as pl
from jax.experimental.pallas import tpu as pltpu
from jax.experimental.pallas import tpu_sc as plsc

N = 10000
E = 320000
D = 128
NEG_SLOPE = 0.2

NC = 2
NS = 16
LANES = 16
HALF = N // NC
AC = 5120
EPT = E // NS
CK = 80
NCHUNK = EPT // CK
RPT = AC // NS
ZR = 160


def _proj_body(x_ref, w_ref, asrc_ref, adst_ref, h_ref, es_ref, ed_ref):
    h = jnp.dot(x_ref[...], w_ref[...], preferred_element_type=jnp.float32)
    h_ref[...] = h
    es_ref[...] = jnp.sum(h * asrc_ref[...][None, :], axis=1)
    ed_ref[...] = jnp.sum(h * adst_ref[...][None, :], axis=1)


def _final_body(acc_ref, den_ref, out_ref):
    a = acc_ref[...]
    den = jnp.sum(den_ref[0], axis=0)[:, None]
    good = den > 0.0
    val = a / jnp.where(good, den, 1.0)
    val = jnp.where(good, val, 0.0)
    out_ref[...] = jnp.where(val > 0.0, val, jnp.exp(val) - 1.0)


def _sc_body(h_hbm, es_hbm, ed_hbm, src_hbm, dst_hbm, acc_hbm, den_hbm,
             es_v, ed_v, den_v, sidx_v, didx_v, gbuf_v, exbuf_v, zbuf_v,
             acc_sh, sem):
    c = lax.axis_index("c")
    s = lax.axis_index("s")

    pltpu.sync_copy(es_hbm, es_v)
    pltpu.sync_copy(ed_hbm, ed_v)

    zero = jnp.zeros((LANES,), jnp.float32)

    @pl.loop(0, AC, step=LANES)
    def _(i):
        den_v[pl.ds(i, LANES)] = zero

    @pl.loop(0, ZR)
    def _(i):
        for v in range(D // LANES):
            zbuf_v[i, pl.ds(v * LANES, LANES)] = zero

    for k in range(RPT // ZR):
        pltpu.sync_copy(zbuf_v, acc_sh.at[pl.ds(s * RPT + k * ZR, ZR)])
    plsc.subcore_barrier()

    lo = c * HALF
    lo16 = jnp.full((LANES,), lo, jnp.int32)
    garbage = jnp.full((LANES,), HALF, jnp.int32) + s
    base = s * EPT

    @pl.loop(0, NCHUNK)
    def _(ci):
        off = base + ci * CK
        pltpu.sync_copy(src_hbm.at[pl.ds(off, CK)], sidx_v)
        pltpu.sync_copy(dst_hbm.at[pl.ds(off, CK)], didx_v)
        pltpu.async_copy(h_hbm.at[sidx_v], gbuf_v, sem).wait()

        @pl.loop(0, CK, step=LANES)
        def _(i):
            s16 = sidx_v[pl.ds(i, LANES)]
            d16 = didx_v[pl.ds(i, LANES)]
            e = plsc.load_gather(es_v, [s16]) + plsc.load_gather(ed_v, [d16])
            e = jnp.where(e > 0.0, e, e * NEG_SLOPE)
            ex = jnp.exp(e)
            exbuf_v[pl.ds(i, LANES)] = ex
            row = d16 - lo16
            local = (row >= 0) & (row < HALF)
            row = jnp.where(local, row, garbage)
            didx_v[pl.ds(i, LANES)] = row
            plsc.addupdate_scatter(den_v, [row], ex)

        @pl.loop(0, CK, step=LANES)
        def _(i):
            exv = exbuf_v[pl.ds(i, LANES)]
            for l in range(LANES):
                scv = jnp.full((LANES,), exv[l], jnp.float32)
                for v in range(D // LANES):
                    gbuf_v[i + l, pl.ds(v * LANES, LANES)] = (
                        gbuf_v[i + l, pl.ds(v * LANES, LANES)] * scv)

        pltpu.sync_copy(gbuf_v, acc_sh.at[didx_v], add=True)

    plsc.subcore_barrier()
    pltpu.sync_copy(acc_sh.at[pl.ds(s * RPT, RPT)],
                    acc_hbm.at[c].at[pl.ds(s * RPT, RPT)])
    pltpu.sync_copy(den_v, den_hbm.at[c].at[s])


@jax.jit
def kernel(x, edge_index, W, a_src, a_dst):
    h, es, ed = pl.pallas_call(
        _proj_body,
        out_shape=(
            jax.ShapeDtypeStruct((N, D), jnp.float32),
            jax.ShapeDtypeStruct((N,), jnp.float32),
            jax.ShapeDtypeStruct((N,), jnp.float32),
        ),
    )(x, W, a_src, a_dst)

    src = edge_index[0].astype(jnp.int32)
    dst = edge_index[1].astype(jnp.int32)

    mesh = plsc.VectorSubcoreMesh(core_axis_name="c", subcore_axis_name="s")
    cp = pltpu.CompilerParams()
    if "needs_layout_passes" in pltpu.CompilerParams.__dataclass_fields__:
        cp = dataclasses.replace(cp, needs_layout_passes=False)
    acc, den = pl.kernel(
        _sc_body,
        out_type=(
            jax.ShapeDtypeStruct((NC, AC, D), jnp.float32),
            jax.ShapeDtypeStruct((NC, NS, AC), jnp.float32),
        ),
        mesh=mesh,
        compiler_params=cp,
        scratch_types=[
            pltpu.VMEM((N,), jnp.float32),
            pltpu.VMEM((N,), jnp.float32),
            pltpu.VMEM((AC,), jnp.float32),
            pltpu.VMEM((CK,), jnp.int32),
            pltpu.VMEM((CK,), jnp.int32),
            pltpu.VMEM((CK, D), jnp.float32),
            pltpu.VMEM((CK,), jnp.float32),
            pltpu.VMEM((ZR, D), jnp.float32),
            pltpu.VMEM_SHARED((AC, D), jnp.float32),
            pltpu.SemaphoreType.DMA,
        ],
    )(h, es, ed, src, dst)

    acc = acc[:, :HALF].reshape(N, D)
    den = jnp.concatenate([den[0, :, :HALF], den[1, :, :HALF]], axis=1)
    den = den.reshape(NS, 10, N // 10).transpose(1, 0, 2)

    out = pl.pallas_call(
        _final_body,
        grid=(10,),
        in_specs=[
            pl.BlockSpec((N // 10, D), lambda i: (i, 0)),
            pl.BlockSpec((1, NS, N // 10), lambda i: (i, 0, 0)),
        ],
        out_specs=pl.BlockSpec((N // 10, D), lambda i: (i, 0)),
        out_shape=jax.ShapeDtypeStruct((N, D), jnp.float32),
    )(acc, den)
    return out
